# Initial kernel scaffold; baseline (speedup 1.0000x reference)
import jax
import jax.numpy as jnp
from jax.experimental import pallas as pl


def kernel(edge_src, edge_dst, edge_rel, edge_batch, query_rel, node_batch, node_ent, rela_embed, W1, W2, Wc, bc):
    raise NotImplementedError("write your pallas kernel here")


# speedup vs baseline: 1.2053x; 1.2053x over previous
import jax
import jax.numpy as jnp
from jax import lax
from jax.experimental import pallas as pl
from jax.experimental.pallas import tpu as pltpu
from jax.experimental.pallas import tpu_sc as plsc

L = 3
E = 320000
N = 10000
B = 32
N_ENT = 10000
D = 20
DH = 10
H = 30
N_EMB = 402
NC = 2
NS = 16
NW = NC * NS
EPT = E // NW
SUP = 2000
NSUP = EPT // SUP
NCH = SUP // 80
GROUPS = 5


def _prep_tables_body(re_ref, qr_ref, w1_ref, w2_ref,
                      tb_ref, tcq_ref, ct0_ref):
    re = re_ref[...]
    w1 = w1_ref[...]
    w2 = w2_ref[...]
    tb = re @ w1[:, D:2 * D].T
    tb_ref[...] = tb
    qoh = (lax.broadcasted_iota(jnp.int32, (B, N_EMB), 1)
           == qr_ref[...][:, None]).astype(jnp.float32)
    qemb = qoh @ re
    tcq = qemb @ w1[:, 2 * D:].T
    tcq_ref[...] = tcq
    a = (tb[:, None, :] + tcq[None, :, :]).reshape(N_EMB * B, H)
    ct0_ref[...] = jax.nn.sigmoid(jnp.maximum(a, 0.0) @ w2.T)


_prep_tables = pl.pallas_call(
    _prep_tables_body,
    out_shape=(
        jax.ShapeDtypeStruct((N_EMB, H), jnp.float32),
        jax.ShapeDtypeStruct((B, H), jnp.float32),
        jax.ShapeDtypeStruct((N_EMB * B, 1), jnp.float32),
    ),
)


def _merge_body(p_ref, h_ref):
    acc = p_ref[0]
    for w in range(1, NW):
        acc = acc + p_ref[w]
    h_ref[...] = acc


_merge = pl.pallas_call(
    _merge_body,
    out_shape=jax.ShapeDtypeStruct((N * DH,), jnp.float32),
)


def _finalize_body(h_ref, wcp_ref, bc_ref, r_ref):
    r_ref[...] = h_ref[...] @ wcp_ref[...] + bc_ref[0]


_finalize = pl.pallas_call(
    _finalize_body,
    in_specs=[
        pl.BlockSpec(memory_space=pltpu.VMEM),
        pl.BlockSpec(memory_space=pltpu.VMEM),
        pl.BlockSpec(memory_space=pltpu.SMEM),
    ],
    out_shape=jax.ShapeDtypeStruct((N, 8), jnp.float32),
)


_SC_MESH = plsc.VectorSubcoreMesh(core_axis_name="c", subcore_axis_name="s")


def _sc_layer0_body(rel_hbm, b_hbm, dst_hbm, ct0f_hbm, rfh_hbm, z_hbm,
                    out_hbm, rel_v, b_v, dst_v, ct0f_v, rfh_v, accum):
    cid = lax.axis_index("c")
    sid = lax.axis_index("s")
    wid = cid * NS + sid
    ebase = wid * EPT
    pltpu.sync_copy(ct0f_hbm, ct0f_v)
    pltpu.sync_copy(rfh_hbm, rfh_v)
    pltpu.sync_copy(z_hbm, accum)

    def sup_body(sc, carry):
        sb = pl.multiple_of(sc * SUP, SUP)
        pltpu.sync_copy(rel_hbm.at[pl.ds(ebase + sb, SUP)], rel_v)
        pltpu.sync_copy(b_hbm.at[pl.ds(ebase + sb, SUP)], b_v)
        pltpu.sync_copy(dst_hbm.at[pl.ds(ebase + sb, SUP)], dst_v)

        def chunk_body(c, carry2):
            cb = pl.multiple_of(c * 80, 80)
            for g in range(GROUPS):
                sl = pl.ds(cb + g * 16, 16)
                relv = rel_v[sl]
                bv = b_v[sl]
                dstv = dst_v[sl]
                s = plsc.load_gather(ct0f_v, [relv * B + bv])
                rb = relv * DH
                db = dstv * DH
                for d in range(DH):
                    rcol = plsc.load_gather(rfh_v, [rb])
                    plsc.addupdate_scatter(accum, [db], s * rcol)
                    rb = rb + 1
                    db = db + 1
            return carry2

        lax.fori_loop(0, NCH, chunk_body, 0)
        return carry

    lax.fori_loop(0, NSUP, sup_body, 0)
    pltpu.sync_copy(accum, out_hbm.at[wid])


_sc_layer0 = pl.kernel(
    _sc_layer0_body,
    out_type=jax.ShapeDtypeStruct((NW, N * DH), jnp.float32),
    mesh=_SC_MESH,
    compiler_params=pltpu.CompilerParams(needs_layout_passes=False,
                                         use_tc_tiling_on_sc=False),
    scratch_types=[
        pltpu.VMEM((SUP,), jnp.int32),
        pltpu.VMEM((SUP,), jnp.int32),
        pltpu.VMEM((SUP,), jnp.int32),
        pltpu.VMEM((N_EMB * B,), jnp.float32),
        pltpu.VMEM((N_EMB * DH,), jnp.float32),
        pltpu.VMEM((N * DH,), jnp.float32),
    ],
)


def kernel(edge_src, edge_dst, edge_rel, edge_batch, query_rel, node_batch,
           node_ent, rela_embed, W1, W2, Wc, bc):
    f32 = jnp.float32
    tb, tcq, ct0 = _prep_tables(rela_embed, query_rel, W1, W2)
    ct0f = ct0.reshape(-1)
    zvec = jnp.zeros((N * DH,), f32)

    halves = []
    for p in range(2):
        rfh = rela_embed[:, p * DH:(p + 1) * DH].reshape(-1)
        part = _sc_layer0(edge_rel[0], edge_batch[0], edge_dst[0], ct0f,
                          rfh, zvec)
        halves.append(_merge(part).reshape(N, DH))
    hidden = jnp.concatenate(halves, axis=1)

    w1a = W1[:, :D]
    w2v = W2[0]
    for i in range(1, L):
        h1 = hidden @ w1a.T
        g = h1[edge_src[i]] + tb[edge_rel[i]] + tcq[edge_batch[i]]
        s = jax.nn.sigmoid(jnp.maximum(g, 0.0) @ w2v)[:, None]
        contrib = s * (hidden[edge_src[i]] + rela_embed[edge_rel[i]])
        hidden = jnp.zeros((N, D), f32).at[edge_dst[i]].add(contrib)

    wcp = jnp.zeros((D, 8), f32).at[:, 0].set(Wc[0])
    res = _finalize(hidden, wcp, bc)
    score_all = jnp.zeros((B, N_ENT), f32).at[node_batch, node_ent].set(
        res[:, 0])
    return score_all
